# baseline (speedup 1.0000x reference)
import functools

import jax
import jax.numpy as jnp
from jax import lax
from jax.experimental import pallas as pl
from jax.experimental.pallas import tpu as pltpu
from jax.experimental.pallas import tpu_sc as plsc

M = 6890
N = 20000
M_PAD = 6912
N_SC = 3616
N_TC = N - N_SC
BN = 2048
LANES = 128
MP = 7168
PT = 224
NV = N_SC // 16
TUF = 8
NC = 2

_mesh = plsc.VectorSubcoreMesh(core_axis_name="c", subcore_axis_name="s")


def _tc_body(scan_ref, temp_ref, out_ref):
    temp2 = temp_ref[:] * 2.0

    def step(j, acc):
        blk = scan_ref[:, pl.ds(j * BN, BN)]
        s0 = blk[0:1, :]
        s1 = blk[1:2, :]
        s2 = blk[2:3, :]
        sqs = s0 * s0 + s1 * s1 + s2 * s2
        cross2 = lax.dot_general(
            temp2, blk, (((1,), (0,)), ((), ())),
            preferred_element_type=jnp.float32)
        v = sqs - cross2
        chunks = [v[:, c * LANES:(c + 1) * LANES] for c in range(BN // LANES)]
        while len(chunks) > 1:
            chunks = [jnp.minimum(chunks[i], chunks[i + 1])
                      for i in range(0, len(chunks), 2)]
        return jnp.minimum(acc, chunks[0])

    acc0 = jnp.full((M_PAD, LANES), jnp.inf, jnp.float32)
    out_ref[:, :] = lax.fori_loop(0, N_TC // BN, step, acc0)


def _rnd_bf16(v):
    c = v * 65537.0
    return c - (c - v)


def _sc_body(sx_hbm, sy_hbm, sz_hbm, tx_hbm, ty_hbm, tz_hbm,
             out_hbm, sx, sy, sz, sq, tq0, tq1, tq2, outv):
    wid = lax.axis_index("s") * NC + lax.axis_index("c")

    pltpu.sync_copy(sx_hbm, sx)
    pltpu.sync_copy(sy_hbm, sy)
    pltpu.sync_copy(sz_hbm, sz)
    base = wid * PT
    pltpu.sync_copy(tx_hbm.at[pl.ds(base, PT)], tq0)
    pltpu.sync_copy(ty_hbm.at[pl.ds(base, PT)], tq1)
    pltpu.sync_copy(tz_hbm.at[pl.ds(base, PT)], tq2)

    def prep_scan(j, _):
        vx = sx[pl.ds(j * 16, 16)]
        vy = sy[pl.ds(j * 16, 16)]
        vz = sz[pl.ds(j * 16, 16)]
        sq[pl.ds(j * 16, 16)] = vx * vx + vy * vy + vz * vz
        sx[pl.ds(j * 16, 16)] = _rnd_bf16(vx)
        sy[pl.ds(j * 16, 16)] = _rnd_bf16(vy)
        sz[pl.ds(j * 16, 16)] = _rnd_bf16(vz)
        return 0
    lax.fori_loop(0, NV, prep_scan, 0)

    def prep_temp(i, _):
        for tqk in (tq0, tq1, tq2):
            v = tqk[pl.ds(i * 16, 16)]
            tqk[pl.ds(i * 16, 16)] = _rnd_bf16(v) * 2.0
        return 0
    lax.fori_loop(0, PT // 16, prep_temp, 0)

    def group(g, _):
        gv = [tqk[pl.ds(g * 16, 16)] for tqk in (tq0, tq1, tq2)]
        for half in range(16 // TUF):
            tb = [[jnp.broadcast_to(gv[k][half * TUF + u], (16,))
                   for k in range(3)] for u in range(TUF)]

            def sweep(j, accs):
                vx = sx[pl.ds(j * 16, 16)]
                vy = sy[pl.ds(j * 16, 16)]
                vz = sz[pl.ds(j * 16, 16)]
                q = sq[pl.ds(j * 16, 16)]
                out = []
                for u in range(TUF):
                    p = tb[u][0] * vx + tb[u][1] * vy + tb[u][2] * vz
                    out.append(jnp.minimum(accs[u], q - p))
                return tuple(out)

            inf = jnp.full((16,), jnp.inf, jnp.float32)
            accs = lax.fori_loop(0, NV, sweep, (inf,) * TUF)
            for u in range(TUF):
                outv[g * 16 + half * TUF + u, :] = accs[u]
        return 0
    lax.fori_loop(0, PT // 16, group, 0)

    pltpu.sync_copy(outv, out_hbm.at[pl.ds(base, PT)])


_sc_call = functools.partial(
    pl.kernel,
    out_type=jax.ShapeDtypeStruct((MP, 16), jnp.float32),
    mesh=_mesh,
    scratch_types=[
        pltpu.VMEM((N_SC,), jnp.float32),
        pltpu.VMEM((N_SC,), jnp.float32),
        pltpu.VMEM((N_SC,), jnp.float32),
        pltpu.VMEM((N_SC,), jnp.float32),
        pltpu.VMEM((PT,), jnp.float32),
        pltpu.VMEM((PT,), jnp.float32),
        pltpu.VMEM((PT,), jnp.float32),
        pltpu.VMEM((PT, 16), jnp.float32),
    ],
)(_sc_body)


def _combine_body(acc_ref, mins_ref, temp_ref, out_ref):
    tsq = temp_ref[:] * temp_ref[:]
    sqt = jnp.sum(tsq, axis=1, keepdims=True)
    m1 = jnp.min(acc_ref[:], axis=1, keepdims=True)
    m2 = jnp.min(mins_ref[0:M_PAD, :], axis=1, keepdims=True)
    dist2 = jnp.minimum(m1, m2) + sqt
    dist2 = jnp.maximum(dist2, 0.0)
    row = lax.broadcasted_iota(jnp.int32, (M_PAD, 1), 0)
    dist2 = jnp.where(row < M, dist2, 0.0)
    out_ref[:, :] = jnp.sum(dist2, keepdims=True)


@functools.partial(jax.jit)
def kernel(scan_vertices, template_vertices):
    temp_pad = jnp.pad(template_vertices, ((0, M_PAD - M), (0, 0)))
    temp_mp = jnp.pad(template_vertices, ((0, MP - M), (0, 0)))
    scan_tc = scan_vertices[:N_TC].T
    acc = pl.pallas_call(
        _tc_body,
        out_shape=jax.ShapeDtypeStruct((M_PAD, LANES), jnp.float32),
    )(scan_tc, temp_pad)
    mins = _sc_call(scan_vertices[N_TC:, 0], scan_vertices[N_TC:, 1],
                    scan_vertices[N_TC:, 2], temp_mp[:, 0],
                    temp_mp[:, 1], temp_mp[:, 2])
    out = pl.pallas_call(
        _combine_body,
        out_shape=jax.ShapeDtypeStruct((1, 1), jnp.float32),
    )(acc, mins, temp_pad)
    return out[0, 0]


# speedup vs baseline: 1.2086x; 1.2086x over previous
import functools

import jax
import jax.numpy as jnp
from jax import lax
from jax.experimental import pallas as pl
from jax.experimental.pallas import tpu as pltpu

M = 6890
N = 20000
M_PAD = 6912
N_PAD = 20480
BN = 1024
UNROLL = 2
LANES = 128
PAD_VAL = 1.0e4


def _chamfer_body(scan_ref, temp_ref, out_ref):
    tsq = temp_ref[:] * temp_ref[:]
    sqt = jnp.sum(tsq, axis=1, keepdims=True)
    temp2 = temp_ref[:] * 2.0

    def one_block(j):
        blk = scan_ref[:, pl.ds(j * BN, BN)]
        s0 = blk[0:1, :]
        s1 = blk[1:2, :]
        s2 = blk[2:3, :]
        sqs = s0 * s0 + s1 * s1 + s2 * s2
        cross2 = lax.dot_general(
            temp2, blk, (((1,), (0,)), ((), ())),
            preferred_element_type=jnp.float32)
        v = sqs - cross2
        chunks = [v[:, c * LANES:(c + 1) * LANES] for c in range(BN // LANES)]
        while len(chunks) > 1:
            chunks = [jnp.minimum(chunks[i], chunks[i + 1])
                      for i in range(0, len(chunks), 2)]
        return chunks[0]

    def step(i, acc):
        parts = [one_block(i * UNROLL + u) for u in range(UNROLL)]
        return jnp.minimum(acc, jnp.minimum(parts[0], parts[1]))

    acc0 = jnp.full((M_PAD, LANES), jnp.inf, jnp.float32)
    acc = lax.fori_loop(0, N_PAD // (BN * UNROLL), step, acc0)

    dist2 = jnp.min(acc, axis=1, keepdims=True) + sqt
    dist2 = jnp.maximum(dist2, 0.0)
    row = lax.broadcasted_iota(jnp.int32, (M_PAD, 1), 0)
    dist2 = jnp.where(row < M, dist2, 0.0)
    out_ref[:, :] = jnp.sum(dist2, keepdims=True)


@functools.partial(jax.jit)
def kernel(scan_vertices, template_vertices):
    scan_t = jnp.pad(scan_vertices, ((0, N_PAD - N), (0, 0)),
                     constant_values=PAD_VAL).T
    temp = jnp.pad(template_vertices, ((0, M_PAD - M), (0, 0)))
    out = pl.pallas_call(
        _chamfer_body,
        out_shape=jax.ShapeDtypeStruct((1, 1), jnp.float32),
    )(scan_t, temp)
    return out[0, 0]
